# Initial kernel scaffold; baseline (speedup 1.0000x reference)
import jax
import jax.numpy as jnp
from jax.experimental import pallas as pl


def kernel(xcorr, nlag):
    raise NotImplementedError("write your pallas kernel here")


# speedup vs baseline: 18.1199x; 18.1199x over previous
import functools

import jax
import jax.numpy as jnp
from jax import lax
from jax.experimental import pallas as pl
from jax.experimental.pallas import tpu as pltpu
from jax.experimental.pallas import tpu_sc as plsc

N = 8192
NT = 1536
OFF = 304
L = 8672
NEG_INF = float("-inf")

_STAGES = ((1, 541), (2, 541), (4, 540), (8, 540),
           (16, 539), (32, 537), (64, 533), (128, 525))


_GDN = lax.GatherDimensionNumbers(
    offset_dims=(), collapsed_slice_dims=(0,), start_index_map=(0,))


def _shuffle(v, idx):
    return lax.gather(v, idx[:, None], dimension_numbers=_GDN,
                      slice_sizes=(1,),
                      mode=lax.GatherScatterMode.PROMISE_IN_BOUNDS)


def _butterfly(v, op, iota16):
    for s in (1, 2, 4, 8):
        v = op(v, _shuffle(v, jnp.bitwise_xor(iota16, s)))
    return v


def _shift_max_pass(src, dst, stride, nchunks):
    def body(c, carry):
        base = 16 * c
        a = src[pl.ds(base, 16)]
        b = src[pl.ds(base + stride, 16)]
        dst[pl.ds(base, 16)] = jnp.maximum(a, b)
        return carry
    lax.fori_loop(0, nchunks, body, 0, unroll=4)


def _peaks_body(xh, out_h, y, buf_a, buf_b, sv):
    nw = 32
    tpw = NT // nw
    wid = lax.axis_index("s") * 2 + lax.axis_index("c")
    t0 = wid * tpw

    minf = jnp.full((16,), NEG_INF, jnp.float32)
    def fill_head(c, carry):
        y[pl.ds(16 * c, 16)] = minf
        return carry
    lax.fori_loop(0, OFF // 16, fill_head, 0)
    def fill_tail(c, carry):
        y[pl.ds(16 * c, 16)] = minf
        return carry
    lax.fori_loop((OFF + N) // 16, L // 16, fill_tail, 0)

    iota16 = lax.broadcasted_iota(jnp.int32, (16,), 0)
    big_i = jnp.full((16,), 2**31 - 1, jnp.int32)

    def do_trace(tt, carry):
        t = t0 + tt
        pltpu.sync_copy(xh.at[pl.ds(t * N, N)], y.at[pl.ds(OFF, N)])

        def abs_body(c, cc):
            base = 16 * c
            y[pl.ds(base, 16)] = jnp.abs(y[pl.ds(base, 16)])
            return cc
        lax.fori_loop(OFF // 16, (OFF + N) // 16, abs_body, 0, unroll=4)

        src, dst = y, buf_a
        for stride, nchunks in _STAGES:
            _shift_max_pass(src, dst, stride, nchunks)
            if dst is buf_a:
                src, dst = buf_a, buf_b
            else:
                src, dst = buf_b, buf_a

        def combine(c, carry):
            b1, i1, b2, i2 = carry
            base = 16 * c
            v = y[pl.ds(base + OFF, 16)]
            m = jnp.maximum(buf_b[pl.ds(base + 154, 16)],
                            buf_b[pl.ds(base + 199, 16)])
            masked = jnp.where(v == m, v, jnp.float32(0.0))
            idxv = iota16 + base
            gt1 = masked > b1
            gt2 = masked > b2
            b2n = jnp.where(gt1, b1, jnp.where(gt2, masked, b2))
            i2n = jnp.where(gt1, i1, jnp.where(gt2, idxv, i2))
            b1n = jnp.where(gt1, masked, b1)
            i1n = jnp.where(gt1, idxv, i1)
            return b1n, i1n, b2n, i2n

        init = (jnp.full((16,), -1.0, jnp.float32), jnp.zeros((16,), jnp.int32),
                jnp.full((16,), -1.0, jnp.float32), jnp.zeros((16,), jnp.int32))
        b1, i1, b2, i2 = lax.fori_loop(0, N // 16, combine, init, unroll=2)

        m1 = _butterfly(b1, jnp.maximum, iota16)
        i1s = _butterfly(jnp.where(b1 == m1, i1, big_i), jnp.minimum, iota16)
        lane_hit = i1 == i1s
        selv = jnp.where(lane_hit, b2, b1)
        seli = jnp.where(lane_hit, i2, i1)
        m2 = _butterfly(selv, jnp.maximum, iota16)
        i2s = _butterfly(jnp.where(selv == m2, seli, big_i), jnp.minimum,
                         iota16)

        packed = jnp.where(
            iota16 < 2, jnp.where(iota16 == 0, m1, m2),
            lax.bitcast_convert_type(jnp.where(iota16 == 2, i1s, i2s),
                                     jnp.float32))
        sv[pl.ds(16 * tt, 16)] = packed
        return carry

    lax.fori_loop(0, tpw, do_trace, 0)

    pltpu.sync_copy(sv, out_h.at[pl.ds(wid * tpw * 16, tpw * 16)])


@jax.jit
def _peaks(x2d):
    mesh = plsc.VectorSubcoreMesh(core_axis_name="c", subcore_axis_name="s")
    tpw = NT // 32
    run = functools.partial(
        pl.kernel,
        mesh=mesh,
        out_type=jax.ShapeDtypeStruct((NT * 16,), jnp.float32),
        scratch_types=[
            pltpu.VMEM((L,), jnp.float32),
            pltpu.VMEM((L,), jnp.float32),
            pltpu.VMEM((L,), jnp.float32),
            pltpu.VMEM((tpw * 16,), jnp.float32),
        ],
    )(_peaks_body)
    return run(x2d)


def kernel(xcorr, nlag):
    nb, nc, nx, n = xcorr.shape
    x1d = xcorr.reshape(nb * nc * nx * n)
    packed = _peaks(x1d).reshape(NT, 16)
    scores = packed[:, 0:2]
    idx = lax.bitcast_convert_type(packed[:, 2:4], jnp.int32)
    return scores.reshape(nb, nc, nx, 2), idx.reshape(nb, nc, nx, 2)
